# Initial kernel scaffold; baseline (speedup 1.0000x reference)
import jax
import jax.numpy as jnp
from jax.experimental import pallas as pl


def kernel(x, h_cur, c_cur, edge_index, W, bias):
    raise NotImplementedError("write your pallas kernel here")


# speedup vs baseline: 17.9765x; 17.9765x over previous
import functools

import jax
import jax.numpy as jnp
from jax import lax
from jax.experimental import pallas as pl
from jax.experimental.pallas import tpu as pltpu
from jax.experimental.pallas import tpu_sc as plsc

B = 2
N = 10000
E = 160000
D_IN = 128
D_H = 64
F = 4 * D_H

NC = 2
NS = 16
NT = NC * NS
K = 128
G = 40
E_PAD = NT * G * K
ACC_ROWS = 10240
ZERO_ROWS = ACC_ROWS // NS
NBLK = 1000
CHUNKS = 4

@functools.lru_cache(maxsize=1)
def _sc_kernels():
    mesh = plsc.VectorSubcoreMesh(core_axis_name="c", subcore_axis_name="s",
                                  num_cores=NC, num_subcores=NS)

    @functools.partial(
        pl.kernel,
        out_type=jax.ShapeDtypeStruct((NC, ACC_ROWS, 128), jnp.float32),
        mesh=mesh,
        scratch_types=[
            pltpu.VMEM((G, K), jnp.int32),
            pltpu.VMEM((K, 128), jnp.float32),
            pltpu.VMEM_SHARED((ACC_ROWS, 128), jnp.float32),
        ],
    )
    def deg_kernel(dst_hbm, ones_hbm, zeros_hbm, out_hbm, dst_idx, ones_v, acc):
        c = lax.axis_index("c")
        s = lax.axis_index("s")
        eb = c * NS + s
        pltpu.sync_copy(dst_hbm.at[eb], dst_idx)
        pltpu.sync_copy(ones_hbm, ones_v)
        pltpu.sync_copy(zeros_hbm, acc.at[pl.ds(s * ZERO_ROWS, ZERO_ROWS)])
        plsc.subcore_barrier()
        for g in range(G):
            pltpu.sync_copy(ones_v, acc.at[dst_idx.at[g]], add=True)
        plsc.subcore_barrier()
        pltpu.sync_copy(acc.at[pl.ds(s * ZERO_ROWS, ZERO_ROWS)],
                        out_hbm.at[c, pl.ds(s * ZERO_ROWS, ZERO_ROWS)])

    @functools.partial(
        pl.kernel,
        out_type=jax.ShapeDtypeStruct((NC, CHUNKS, ACC_ROWS, 128), jnp.float32),
        mesh=mesh,
        scratch_types=[
            pltpu.VMEM((CHUNKS, G, K), jnp.int32),
            pltpu.VMEM((G, K), jnp.int32),
            pltpu.VMEM((K, 128), jnp.float32),
            pltpu.VMEM_SHARED((ACC_ROWS, 128), jnp.float32),
            pltpu.SemaphoreType.DMA,
        ],
    )
    def mp_kernel(table_hbm, src_hbm, dst_hbm, zeros_hbm, out_hbm,
                  src_idx, dst_idx, buf, acc, sem):
        c = lax.axis_index("c")
        s = lax.axis_index("s")
        eb = c * NS + s
        pltpu.sync_copy(src_hbm.at[eb], src_idx)
        pltpu.sync_copy(dst_hbm.at[eb], dst_idx)
        for ch in range(CHUNKS):
            pltpu.sync_copy(zeros_hbm, acc.at[pl.ds(s * ZERO_ROWS, ZERO_ROWS)])
            plsc.subcore_barrier()

            @pl.loop(0, G)
            def _(g):
                pltpu.async_copy(table_hbm.at[src_idx.at[ch, g]], buf, sem).wait()
                pltpu.sync_copy(buf, acc.at[dst_idx.at[g]], add=True)

            plsc.subcore_barrier()
            pltpu.sync_copy(acc.at[pl.ds(s * ZERO_ROWS, ZERO_ROWS)],
                            out_hbm.at[c, ch, pl.ds(s * ZERO_ROWS, ZERO_ROWS)])
            plsc.subcore_barrier()

    return deg_kernel, mp_kernel


def _tc1_body(x_ref, h_ref, w_ref, dp_ref, out_ref):
    xb = x_ref[0]
    hb = h_ref[0]
    xw = (jnp.dot(xb, w_ref[0:D_IN, :], preferred_element_type=jnp.float32)
          + jnp.dot(hb, w_ref[D_IN:, :], preferred_element_type=jnp.float32))
    deg = dp_ref[0, :, 0] + dp_ref[1, :, 0] + 1.0
    dinv = lax.rsqrt(deg)
    scaled = xw * dinv[:, None]
    out_ref[0] = scaled[:, 0:128]
    out_ref[1] = scaled[:, 128:256]


def _tc1(x, h_cur, w, deg_parts):
    return pl.pallas_call(
        _tc1_body,
        grid=(B, N // NBLK),
        in_specs=[
            pl.BlockSpec((1, NBLK, D_IN), lambda b, n: (b, n, 0)),
            pl.BlockSpec((1, NBLK, D_H), lambda b, n: (b, n, 0)),
            pl.BlockSpec((D_IN + D_H, F), lambda b, n: (0, 0)),
            pl.BlockSpec((NC, NBLK, 128), lambda b, n: (0, n, 0)),
        ],
        out_specs=pl.BlockSpec((2, NBLK, 128), lambda b, n: (b, n, 0)),
        out_shape=jax.ShapeDtypeStruct((CHUNKS, N, 128), jnp.float32),
    )(x, h_cur, w, deg_parts)


def _tc2_body(parts_ref, table_ref, dp_ref, c_ref, bias_ref, h_out, c_out):
    s0 = parts_ref[0, 0] + parts_ref[1, 0] + table_ref[0]
    s1 = parts_ref[0, 1] + parts_ref[1, 1] + table_ref[1]
    deg = dp_ref[0, :, 0] + dp_ref[1, :, 0] + 1.0
    dinv = lax.rsqrt(deg)
    a0 = s0 * dinv[:, None] + bias_ref[0:1, :]
    a1 = s1 * dinv[:, None] + bias_ref[1:2, :]
    gi = jax.nn.sigmoid(a0[:, 0:D_H])
    gf = jax.nn.sigmoid(a0[:, D_H:2 * D_H])
    go = jax.nn.sigmoid(a1[:, 0:D_H])
    gg = jnp.tanh(a1[:, D_H:2 * D_H])
    cn = gf * c_ref[0] + gi * gg
    c_out[0] = cn
    h_out[0] = go * jnp.tanh(cn)


def _tc2(parts, table, deg_parts, c_cur, bias2):
    return pl.pallas_call(
        _tc2_body,
        grid=(B, N // NBLK),
        in_specs=[
            pl.BlockSpec((NC, 2, NBLK, 128), lambda b, n: (0, b, n, 0)),
            pl.BlockSpec((2, NBLK, 128), lambda b, n: (b, n, 0)),
            pl.BlockSpec((NC, NBLK, 128), lambda b, n: (0, n, 0)),
            pl.BlockSpec((1, NBLK, D_H), lambda b, n: (b, n, 0)),
            pl.BlockSpec((2, 128), lambda b, n: (0, 0)),
        ],
        out_specs=[
            pl.BlockSpec((1, NBLK, D_H), lambda b, n: (b, n, 0)),
            pl.BlockSpec((1, NBLK, D_H), lambda b, n: (b, n, 0)),
        ],
        out_shape=[
            jax.ShapeDtypeStruct((B, N, D_H), jnp.float32),
            jax.ShapeDtypeStruct((B, N, D_H), jnp.float32),
        ],
    )(parts, table, deg_parts, c_cur, bias2)


def kernel(x, h_cur, c_cur, edge_index, W, bias):
    src = edge_index[0]
    dst = edge_index[1]
    npad = E_PAD - E
    src_p = jnp.concatenate([src, jnp.zeros((npad,), jnp.int32)])
    dst_p = jnp.concatenate([dst, jnp.full((npad,), N, jnp.int32)])
    dst_t = dst_p.reshape(NT, G, K)
    offs = jnp.arange(CHUNKS, dtype=jnp.int32) * N
    src4 = (src_p[None, :] + offs[:, None]).reshape(CHUNKS, NT, G, K)
    src4 = src4.transpose(1, 0, 2, 3)

    ones128 = jnp.ones((K, 128), jnp.float32)
    zeros128 = jnp.zeros((ZERO_ROWS, 128), jnp.float32)

    deg_kernel, mp_kernel = _sc_kernels()
    deg_parts = deg_kernel(dst_t, ones128, zeros128)
    table = _tc1(x, h_cur, W, deg_parts)
    parts = mp_kernel(table.reshape(CHUNKS * N, 128), src4, dst_t, zeros128)
    h_next, c_next = _tc2(parts, table, deg_parts, c_cur, bias.reshape(2, 128))
    return (h_next, c_next)


# speedup vs baseline: 20.0230x; 1.1138x over previous
import functools

import jax
import jax.numpy as jnp
from jax import lax
from jax.experimental import pallas as pl
from jax.experimental.pallas import tpu as pltpu
from jax.experimental.pallas import tpu_sc as plsc

B = 2
N = 10000
E = 160000
D_IN = 128
D_H = 64
F = 4 * D_H

NC = 2
NS = 16
NT = NC * NS
K = 128
G = 40
E_PAD = NT * G * K
ACC_ROWS = 10240
ZERO_ROWS = ACC_ROWS // NS
NBLK = 1000
CHUNKS = 4

@functools.lru_cache(maxsize=1)
def _sc_kernels():
    mesh = plsc.VectorSubcoreMesh(core_axis_name="c", subcore_axis_name="s",
                                  num_cores=NC, num_subcores=NS)

    @functools.partial(
        pl.kernel,
        out_type=jax.ShapeDtypeStruct((NC, ACC_ROWS, 128), jnp.float32),
        mesh=mesh,
        scratch_types=[
            pltpu.VMEM((G, K), jnp.int32),
            pltpu.VMEM((K, 128), jnp.float32),
            pltpu.VMEM_SHARED((ACC_ROWS, 128), jnp.float32),
        ],
    )
    def deg_kernel(dst_hbm, ones_hbm, zeros_hbm, out_hbm, dst_idx, ones_v, acc):
        c = lax.axis_index("c")
        s = lax.axis_index("s")
        eb = c * NS + s
        pltpu.sync_copy(dst_hbm.at[eb], dst_idx)
        pltpu.sync_copy(ones_hbm, ones_v)
        pltpu.sync_copy(zeros_hbm, acc.at[pl.ds(s * ZERO_ROWS, ZERO_ROWS)])
        plsc.subcore_barrier()
        for g in range(G):
            pltpu.sync_copy(ones_v, acc.at[dst_idx.at[g]], add=True)
        plsc.subcore_barrier()
        pltpu.sync_copy(acc.at[pl.ds(s * ZERO_ROWS, ZERO_ROWS)],
                        out_hbm.at[c, pl.ds(s * ZERO_ROWS, ZERO_ROWS)])

    @functools.partial(
        pl.kernel,
        out_type=jax.ShapeDtypeStruct((NC, CHUNKS, ACC_ROWS, 128), jnp.float32),
        mesh=mesh,
        scratch_types=[
            pltpu.VMEM((G, K), jnp.int32),
            pltpu.VMEM((G, K), jnp.int32),
            pltpu.VMEM((K, 128), jnp.float32),
            pltpu.VMEM((K, 128), jnp.float32),
            pltpu.VMEM_SHARED((ACC_ROWS, 128), jnp.float32),
            pltpu.SemaphoreType.DMA,
            pltpu.SemaphoreType.DMA,
        ],
    )
    def mp_kernel(table_hbm, src_hbm, dst_hbm, zeros_hbm, out_hbm,
                  src_idx, dst_idx, buf0, buf1, acc, sem0, sem1):
        c = lax.axis_index("c")
        s = lax.axis_index("s")
        eb = c * NS + s
        pltpu.sync_copy(dst_hbm.at[eb], dst_idx)
        for ch in range(CHUNKS):
            pltpu.sync_copy(src_hbm.at[eb, ch], src_idx)
            pltpu.sync_copy(zeros_hbm, acc.at[pl.ds(s * ZERO_ROWS, ZERO_ROWS)])
            plsc.subcore_barrier()

            pltpu.async_copy(table_hbm.at[src_idx.at[0]], buf0, sem0)
            pltpu.async_copy(table_hbm.at[src_idx.at[1]], buf1, sem1)

            @pl.loop(0, G - 2, step=2)
            def _(g):
                pltpu.make_async_copy(table_hbm.at[src_idx.at[0]],
                                      buf0, sem0).wait()
                pltpu.sync_copy(buf0, acc.at[dst_idx.at[g]], add=True)
                pltpu.async_copy(table_hbm.at[src_idx.at[g + 2]], buf0, sem0)
                pltpu.make_async_copy(table_hbm.at[src_idx.at[1]],
                                      buf1, sem1).wait()
                pltpu.sync_copy(buf1, acc.at[dst_idx.at[g + 1]], add=True)
                pltpu.async_copy(table_hbm.at[src_idx.at[g + 3]], buf1, sem1)

            pltpu.make_async_copy(table_hbm.at[src_idx.at[0]],
                                  buf0, sem0).wait()
            pltpu.sync_copy(buf0, acc.at[dst_idx.at[G - 2]], add=True)
            pltpu.make_async_copy(table_hbm.at[src_idx.at[1]],
                                  buf1, sem1).wait()
            pltpu.sync_copy(buf1, acc.at[dst_idx.at[G - 1]], add=True)

            plsc.subcore_barrier()
            pltpu.sync_copy(acc.at[pl.ds(s * ZERO_ROWS, ZERO_ROWS)],
                            out_hbm.at[c, ch, pl.ds(s * ZERO_ROWS, ZERO_ROWS)])
            plsc.subcore_barrier()

    return deg_kernel, mp_kernel


def _tc1_body(x_ref, h_ref, w_ref, dp_ref, out_ref):
    xb = x_ref[0]
    hb = h_ref[0]
    xw = (jnp.dot(xb, w_ref[0:D_IN, :], preferred_element_type=jnp.float32)
          + jnp.dot(hb, w_ref[D_IN:, :], preferred_element_type=jnp.float32))
    deg = dp_ref[0, :, 0] + dp_ref[1, :, 0] + 1.0
    dinv = lax.rsqrt(deg)
    scaled = xw * dinv[:, None]
    out_ref[0] = scaled[:, 0:128]
    out_ref[1] = scaled[:, 128:256]


def _tc1(x, h_cur, w, deg_parts):
    return pl.pallas_call(
        _tc1_body,
        grid=(B, N // NBLK),
        in_specs=[
            pl.BlockSpec((1, NBLK, D_IN), lambda b, n: (b, n, 0)),
            pl.BlockSpec((1, NBLK, D_H), lambda b, n: (b, n, 0)),
            pl.BlockSpec((D_IN + D_H, F), lambda b, n: (0, 0)),
            pl.BlockSpec((NC, NBLK, 128), lambda b, n: (0, n, 0)),
        ],
        out_specs=pl.BlockSpec((2, NBLK, 128), lambda b, n: (b, n, 0)),
        out_shape=jax.ShapeDtypeStruct((CHUNKS, N, 128), jnp.float32),
    )(x, h_cur, w, deg_parts)


def _tc2_body(parts_ref, table_ref, dp_ref, c_ref, bias_ref, h_out, c_out):
    s0 = parts_ref[0, 0] + parts_ref[1, 0] + table_ref[0]
    s1 = parts_ref[0, 1] + parts_ref[1, 1] + table_ref[1]
    deg = dp_ref[0, :, 0] + dp_ref[1, :, 0] + 1.0
    dinv = lax.rsqrt(deg)
    a0 = s0 * dinv[:, None] + bias_ref[0:1, :]
    a1 = s1 * dinv[:, None] + bias_ref[1:2, :]
    gi = jax.nn.sigmoid(a0[:, 0:D_H])
    gf = jax.nn.sigmoid(a0[:, D_H:2 * D_H])
    go = jax.nn.sigmoid(a1[:, 0:D_H])
    gg = jnp.tanh(a1[:, D_H:2 * D_H])
    cn = gf * c_ref[0] + gi * gg
    c_out[0] = cn
    h_out[0] = go * jnp.tanh(cn)


def _tc2(parts, table, deg_parts, c_cur, bias2):
    return pl.pallas_call(
        _tc2_body,
        grid=(B, N // NBLK),
        in_specs=[
            pl.BlockSpec((NC, 2, NBLK, 128), lambda b, n: (0, b, n, 0)),
            pl.BlockSpec((2, NBLK, 128), lambda b, n: (b, n, 0)),
            pl.BlockSpec((NC, NBLK, 128), lambda b, n: (0, n, 0)),
            pl.BlockSpec((1, NBLK, D_H), lambda b, n: (b, n, 0)),
            pl.BlockSpec((2, 128), lambda b, n: (0, 0)),
        ],
        out_specs=[
            pl.BlockSpec((1, NBLK, D_H), lambda b, n: (b, n, 0)),
            pl.BlockSpec((1, NBLK, D_H), lambda b, n: (b, n, 0)),
        ],
        out_shape=[
            jax.ShapeDtypeStruct((B, N, D_H), jnp.float32),
            jax.ShapeDtypeStruct((B, N, D_H), jnp.float32),
        ],
    )(parts, table, deg_parts, c_cur, bias2)


def kernel(x, h_cur, c_cur, edge_index, W, bias):
    src = edge_index[0]
    dst = edge_index[1]
    npad = E_PAD - E
    src_p = jnp.concatenate([src, jnp.zeros((npad,), jnp.int32)])
    dst_p = jnp.concatenate([dst, jnp.full((npad,), N, jnp.int32)])
    dst_t = dst_p.reshape(NT, G, K)
    offs = jnp.arange(CHUNKS, dtype=jnp.int32) * N
    src4 = (src_p[None, :] + offs[:, None]).reshape(CHUNKS, NT, G, K)
    src4 = src4.transpose(1, 0, 2, 3)

    ones128 = jnp.ones((K, 128), jnp.float32)
    zeros128 = jnp.zeros((ZERO_ROWS, 128), jnp.float32)

    deg_kernel, mp_kernel = _sc_kernels()
    deg_parts = deg_kernel(dst_t, ones128, zeros128)
    table = _tc1(x, h_cur, W, deg_parts)
    parts = mp_kernel(table.reshape(CHUNKS * N, 128), src4, dst_t, zeros128)
    h_next, c_next = _tc2(parts, table, deg_parts, c_cur, bias.reshape(2, 128))
    return (h_next, c_next)


# speedup vs baseline: 21.5020x; 1.0739x over previous
import functools

import jax
import jax.numpy as jnp
from jax import lax
from jax.experimental import pallas as pl
from jax.experimental.pallas import tpu as pltpu
from jax.experimental.pallas import tpu_sc as plsc

B = 2
N = 10000
E = 160000
D_IN = 128
D_H = 64
F = 4 * D_H

NC = 2
NS = 16
NT = NC * NS
K = 64
G = 80
NBUF = 4
E_PAD = NT * G * K
ACC_ROWS = 10112
ZERO_ROWS = ACC_ROWS // NS
NBLK = 1000
CHUNKS = 4

@functools.lru_cache(maxsize=1)
def _sc_kernels():
    mesh = plsc.VectorSubcoreMesh(core_axis_name="c", subcore_axis_name="s",
                                  num_cores=NC, num_subcores=NS)

    @functools.partial(
        pl.kernel,
        out_type=jax.ShapeDtypeStruct((NC, ACC_ROWS, 128), jnp.float32),
        mesh=mesh,
        scratch_types=[
            pltpu.VMEM((G, K), jnp.int32),
            pltpu.VMEM((K, 128), jnp.float32),
            pltpu.VMEM_SHARED((ACC_ROWS, 128), jnp.float32),
        ],
    )
    def deg_kernel(dst_hbm, ones_hbm, zeros_hbm, out_hbm, dst_idx, ones_v, acc):
        c = lax.axis_index("c")
        s = lax.axis_index("s")
        eb = c * NS + s
        pltpu.sync_copy(dst_hbm.at[eb], dst_idx)
        pltpu.sync_copy(ones_hbm, ones_v)
        pltpu.sync_copy(zeros_hbm, acc.at[pl.ds(s * ZERO_ROWS, ZERO_ROWS)])
        plsc.subcore_barrier()
        for g in range(G):
            pltpu.sync_copy(ones_v, acc.at[dst_idx.at[g]], add=True)
        plsc.subcore_barrier()
        pltpu.sync_copy(acc.at[pl.ds(s * ZERO_ROWS, ZERO_ROWS)],
                        out_hbm.at[c, pl.ds(s * ZERO_ROWS, ZERO_ROWS)])

    @functools.partial(
        pl.kernel,
        out_type=jax.ShapeDtypeStruct((NC, CHUNKS, ACC_ROWS, 128), jnp.float32),
        mesh=mesh,
        scratch_types=[
            pltpu.VMEM((G * K,), jnp.int32),
            pltpu.VMEM((G, K), jnp.int32),
        ] + [pltpu.VMEM((K, 128), jnp.float32) for _ in range(NBUF)]
          + [pltpu.VMEM_SHARED((ACC_ROWS, 128), jnp.float32)]
          + [pltpu.SemaphoreType.DMA for _ in range(NBUF)],
    )
    def mp_kernel(table_hbm, src_hbm, dst_hbm, zeros_hbm, out_hbm,
                  src_idx, dst_idx, *rest):
        bufs = rest[:NBUF]
        acc = rest[NBUF]
        sems = rest[NBUF + 1:]
        c = lax.axis_index("c")
        s = lax.axis_index("s")
        eb = c * NS + s
        pltpu.sync_copy(dst_hbm.at[eb], dst_idx)
        for ch in range(CHUNKS):
            pltpu.sync_copy(src_hbm.at[eb, ch], src_idx)
            pltpu.sync_copy(zeros_hbm, acc.at[pl.ds(s * ZERO_ROWS, ZERO_ROWS)])
            plsc.subcore_barrier()

            for b in range(NBUF):
                pltpu.async_copy(table_hbm.at[src_idx.at[pl.ds(b * K, K)]],
                                 bufs[b], sems[b])

            @pl.loop(0, G - NBUF, step=NBUF)
            def _(g):
                for b in range(NBUF):
                    pltpu.make_async_copy(table_hbm.at[src_idx.at[pl.ds(0, K)]],
                                          bufs[b], sems[b]).wait()
                    pltpu.sync_copy(bufs[b], acc.at[dst_idx.at[g + b]], add=True)
                    pltpu.async_copy(
                        table_hbm.at[src_idx.at[pl.ds((g + NBUF + b) * K, K)]],
                        bufs[b], sems[b])

            for b in range(NBUF):
                pltpu.make_async_copy(table_hbm.at[src_idx.at[pl.ds(0, K)]],
                                      bufs[b], sems[b]).wait()
                pltpu.sync_copy(bufs[b], acc.at[dst_idx.at[G - NBUF + b]],
                                add=True)

            plsc.subcore_barrier()
            pltpu.sync_copy(acc.at[pl.ds(s * ZERO_ROWS, ZERO_ROWS)],
                            out_hbm.at[c, ch, pl.ds(s * ZERO_ROWS, ZERO_ROWS)])
            plsc.subcore_barrier()

    return deg_kernel, mp_kernel


def _tc1_body(x_ref, h_ref, w_ref, dp_ref, out_ref):
    xb = x_ref[0]
    hb = h_ref[0]
    xw = (jnp.dot(xb, w_ref[0:D_IN, :], preferred_element_type=jnp.float32)
          + jnp.dot(hb, w_ref[D_IN:, :], preferred_element_type=jnp.float32))
    deg = dp_ref[0, :, 0] + dp_ref[1, :, 0] + 1.0
    dinv = lax.rsqrt(deg)
    scaled = xw * dinv[:, None]
    out_ref[0] = scaled[:, 0:128]
    out_ref[1] = scaled[:, 128:256]


def _tc1(x, h_cur, w, deg_parts):
    return pl.pallas_call(
        _tc1_body,
        grid=(B, N // NBLK),
        in_specs=[
            pl.BlockSpec((1, NBLK, D_IN), lambda b, n: (b, n, 0)),
            pl.BlockSpec((1, NBLK, D_H), lambda b, n: (b, n, 0)),
            pl.BlockSpec((D_IN + D_H, F), lambda b, n: (0, 0)),
            pl.BlockSpec((NC, NBLK, 128), lambda b, n: (0, n, 0)),
        ],
        out_specs=pl.BlockSpec((2, NBLK, 128), lambda b, n: (b, n, 0)),
        out_shape=jax.ShapeDtypeStruct((CHUNKS, N, 128), jnp.float32),
    )(x, h_cur, w, deg_parts)


def _tc2_body(parts_ref, table_ref, dp_ref, c_ref, bias_ref, h_out, c_out):
    s0 = parts_ref[0, 0] + parts_ref[1, 0] + table_ref[0]
    s1 = parts_ref[0, 1] + parts_ref[1, 1] + table_ref[1]
    deg = dp_ref[0, :, 0] + dp_ref[1, :, 0] + 1.0
    dinv = lax.rsqrt(deg)
    a0 = s0 * dinv[:, None] + bias_ref[0:1, :]
    a1 = s1 * dinv[:, None] + bias_ref[1:2, :]
    gi = jax.nn.sigmoid(a0[:, 0:D_H])
    gf = jax.nn.sigmoid(a0[:, D_H:2 * D_H])
    go = jax.nn.sigmoid(a1[:, 0:D_H])
    gg = jnp.tanh(a1[:, D_H:2 * D_H])
    cn = gf * c_ref[0] + gi * gg
    c_out[0] = cn
    h_out[0] = go * jnp.tanh(cn)


def _tc2(parts, table, deg_parts, c_cur, bias2):
    return pl.pallas_call(
        _tc2_body,
        grid=(B, N // NBLK),
        in_specs=[
            pl.BlockSpec((NC, 2, NBLK, 128), lambda b, n: (0, b, n, 0)),
            pl.BlockSpec((2, NBLK, 128), lambda b, n: (b, n, 0)),
            pl.BlockSpec((NC, NBLK, 128), lambda b, n: (0, n, 0)),
            pl.BlockSpec((1, NBLK, D_H), lambda b, n: (b, n, 0)),
            pl.BlockSpec((2, 128), lambda b, n: (0, 0)),
        ],
        out_specs=[
            pl.BlockSpec((1, NBLK, D_H), lambda b, n: (b, n, 0)),
            pl.BlockSpec((1, NBLK, D_H), lambda b, n: (b, n, 0)),
        ],
        out_shape=[
            jax.ShapeDtypeStruct((B, N, D_H), jnp.float32),
            jax.ShapeDtypeStruct((B, N, D_H), jnp.float32),
        ],
    )(parts, table, deg_parts, c_cur, bias2)


def kernel(x, h_cur, c_cur, edge_index, W, bias):
    src = edge_index[0]
    dst = edge_index[1]
    npad = E_PAD - E
    src_p = jnp.concatenate([src, jnp.zeros((npad,), jnp.int32)])
    dst_p = jnp.concatenate([dst, jnp.full((npad,), N, jnp.int32)])
    dst_t = dst_p.reshape(NT, G, K)
    offs = jnp.arange(CHUNKS, dtype=jnp.int32) * N
    src4 = (src_p[None, :] + offs[:, None]).reshape(CHUNKS, NT, G * K)
    src4 = src4.transpose(1, 0, 2)

    ones128 = jnp.ones((K, 128), jnp.float32)
    zeros128 = jnp.zeros((ZERO_ROWS, 128), jnp.float32)

    deg_kernel, mp_kernel = _sc_kernels()
    deg_parts = deg_kernel(dst_t, ones128, zeros128)
    table = _tc1(x, h_cur, W, deg_parts)
    parts = mp_kernel(table.reshape(CHUNKS * N, 128), src4, dst_t, zeros128)
    h_next, c_next = _tc2(parts, table, deg_parts, c_cur, bias.reshape(2, 128))
    return (h_next, c_next)


# speedup vs baseline: 25.1725x; 1.1707x over previous
import functools

import jax
import jax.numpy as jnp
from jax import lax
from jax.experimental import pallas as pl
from jax.experimental.pallas import tpu as pltpu
from jax.experimental.pallas import tpu_sc as plsc

B = 2
N = 10000
E = 160000
D_IN = 128
D_H = 64
F = 4 * D_H

NC = 2
NS = 16
NT = NC * NS
K = 64
NBUF = 3
G0 = 123
G1 = 36
GD = 80
E0 = NS * G0 * K
E1 = NS * G1 * K
E_PAD = E0 + E1
E_PAD_D = NT * GD * K
ACC_ROWS = 10112
ZERO_ROWS = ACC_ROWS // NS
NBLK = 1000
CHUNKS = 4

@functools.lru_cache(maxsize=1)
def _sc_kernels():
    mesh = plsc.VectorSubcoreMesh(core_axis_name="c", subcore_axis_name="s",
                                  num_cores=NC, num_subcores=NS)

    @functools.partial(
        pl.kernel,
        out_type=jax.ShapeDtypeStruct((NC, ACC_ROWS, 128), jnp.float32),
        mesh=mesh,
        scratch_types=[
            pltpu.VMEM((GD, K), jnp.int32),
            pltpu.VMEM((K, 128), jnp.float32),
            pltpu.VMEM_SHARED((ACC_ROWS, 128), jnp.float32),
        ],
    )
    def deg_kernel(dst_hbm, ones_hbm, zeros_hbm, out_hbm, dst_idx, ones_v, acc):
        c = lax.axis_index("c")
        s = lax.axis_index("s")
        eb = c * NS + s
        pltpu.sync_copy(dst_hbm.at[eb], dst_idx)
        pltpu.sync_copy(ones_hbm, ones_v)
        pltpu.sync_copy(zeros_hbm, acc.at[pl.ds(s * ZERO_ROWS, ZERO_ROWS)])
        plsc.subcore_barrier()
        for g in range(GD):
            pltpu.sync_copy(ones_v, acc.at[dst_idx.at[g]], add=True)
        plsc.subcore_barrier()
        pltpu.sync_copy(acc.at[pl.ds(s * ZERO_ROWS, ZERO_ROWS)],
                        out_hbm.at[c, pl.ds(s * ZERO_ROWS, ZERO_ROWS)])

    @functools.partial(
        pl.kernel,
        out_type=jax.ShapeDtypeStruct((NC, CHUNKS, ACC_ROWS, 128), jnp.float32),
        mesh=mesh,
        scratch_types=[
            pltpu.VMEM((G0 * K,), jnp.int32),
            pltpu.VMEM((G0, K), jnp.int32),
        ] + [pltpu.VMEM((K, 128), jnp.float32) for _ in range(NBUF)]
          + [pltpu.VMEM_SHARED((ACC_ROWS, 128), jnp.float32)]
          + [pltpu.SemaphoreType.DMA for _ in range(NBUF)],
    )
    def mp_kernel(table_hbm, src_hbm, dst_hbm, zeros_hbm, out_hbm,
                  src_idx, dst_idx, *rest):
        bufs = rest[:NBUF]
        acc = rest[NBUF]
        sems = rest[NBUF + 1:]
        c = lax.axis_index("c")
        s = lax.axis_index("s")
        eb = c * NS + s
        ng = jnp.where(c == 0, G0, G1)
        pltpu.sync_copy(dst_hbm.at[eb], dst_idx)
        for ch in range(CHUNKS):
            pltpu.sync_copy(src_hbm.at[eb, ch], src_idx)
            pltpu.sync_copy(zeros_hbm, acc.at[pl.ds(s * ZERO_ROWS, ZERO_ROWS)])
            plsc.subcore_barrier()

            for b in range(NBUF):
                pltpu.async_copy(table_hbm.at[src_idx.at[pl.ds(b * K, K)]],
                                 bufs[b], sems[b])

            @pl.loop(0, ng - NBUF, step=NBUF)
            def _(g):
                for b in range(NBUF):
                    pltpu.make_async_copy(table_hbm.at[src_idx.at[pl.ds(0, K)]],
                                          bufs[b], sems[b]).wait()
                    pltpu.sync_copy(bufs[b], acc.at[dst_idx.at[g + b]], add=True)
                    pltpu.async_copy(
                        table_hbm.at[src_idx.at[pl.ds((g + NBUF + b) * K, K)]],
                        bufs[b], sems[b])

            for b in range(NBUF):
                pltpu.make_async_copy(table_hbm.at[src_idx.at[pl.ds(0, K)]],
                                      bufs[b], sems[b]).wait()
                pltpu.sync_copy(bufs[b], acc.at[dst_idx.at[ng - NBUF + b]],
                                add=True)

            plsc.subcore_barrier()
            pltpu.sync_copy(acc.at[pl.ds(s * ZERO_ROWS, ZERO_ROWS)],
                            out_hbm.at[c, ch, pl.ds(s * ZERO_ROWS, ZERO_ROWS)])
            plsc.subcore_barrier()

    return deg_kernel, mp_kernel


def _tc1_body(x_ref, h_ref, w_ref, dp_ref, out_ref):
    xb = x_ref[0]
    hb = h_ref[0]
    xw = (jnp.dot(xb, w_ref[0:D_IN, :], preferred_element_type=jnp.float32)
          + jnp.dot(hb, w_ref[D_IN:, :], preferred_element_type=jnp.float32))
    deg = dp_ref[0, :, 0] + dp_ref[1, :, 0] + 1.0
    dinv = lax.rsqrt(deg)
    scaled = xw * dinv[:, None]
    out_ref[0] = scaled[:, 0:128]
    out_ref[1] = scaled[:, 128:256]


def _tc1(x, h_cur, w, deg_parts):
    return pl.pallas_call(
        _tc1_body,
        grid=(B, N // NBLK),
        in_specs=[
            pl.BlockSpec((1, NBLK, D_IN), lambda b, n: (b, n, 0)),
            pl.BlockSpec((1, NBLK, D_H), lambda b, n: (b, n, 0)),
            pl.BlockSpec((D_IN + D_H, F), lambda b, n: (0, 0)),
            pl.BlockSpec((NC, NBLK, 128), lambda b, n: (0, n, 0)),
        ],
        out_specs=pl.BlockSpec((2, NBLK, 128), lambda b, n: (b, n, 0)),
        out_shape=jax.ShapeDtypeStruct((CHUNKS, N, 128), jnp.float32),
    )(x, h_cur, w, deg_parts)


def _tc2_body(parts_ref, table_ref, dp_ref, c_ref, bias_ref, h_out, c_out):
    s0 = parts_ref[0, 0] + parts_ref[1, 0] + table_ref[0]
    s1 = parts_ref[0, 1] + parts_ref[1, 1] + table_ref[1]
    deg = dp_ref[0, :, 0] + dp_ref[1, :, 0] + 1.0
    dinv = lax.rsqrt(deg)
    a0 = s0 * dinv[:, None] + bias_ref[0:1, :]
    a1 = s1 * dinv[:, None] + bias_ref[1:2, :]
    gi = jax.nn.sigmoid(a0[:, 0:D_H])
    gf = jax.nn.sigmoid(a0[:, D_H:2 * D_H])
    go = jax.nn.sigmoid(a1[:, 0:D_H])
    gg = jnp.tanh(a1[:, D_H:2 * D_H])
    cn = gf * c_ref[0] + gi * gg
    c_out[0] = cn
    h_out[0] = go * jnp.tanh(cn)


def _tc2(parts, table, deg_parts, c_cur, bias2):
    return pl.pallas_call(
        _tc2_body,
        grid=(B, N // NBLK),
        in_specs=[
            pl.BlockSpec((NC, 2, NBLK, 128), lambda b, n: (0, b, n, 0)),
            pl.BlockSpec((2, NBLK, 128), lambda b, n: (b, n, 0)),
            pl.BlockSpec((NC, NBLK, 128), lambda b, n: (0, n, 0)),
            pl.BlockSpec((1, NBLK, D_H), lambda b, n: (b, n, 0)),
            pl.BlockSpec((2, 128), lambda b, n: (0, 0)),
        ],
        out_specs=[
            pl.BlockSpec((1, NBLK, D_H), lambda b, n: (b, n, 0)),
            pl.BlockSpec((1, NBLK, D_H), lambda b, n: (b, n, 0)),
        ],
        out_shape=[
            jax.ShapeDtypeStruct((B, N, D_H), jnp.float32),
            jax.ShapeDtypeStruct((B, N, D_H), jnp.float32),
        ],
    )(parts, table, deg_parts, c_cur, bias2)


def kernel(x, h_cur, c_cur, edge_index, W, bias):
    src = edge_index[0]
    dst = edge_index[1]
    npad = E_PAD - E
    src_p = jnp.concatenate([src, jnp.zeros((npad,), jnp.int32)])
    dst_p = jnp.concatenate([dst, jnp.full((npad,), N, jnp.int32)])
    src_t = jnp.concatenate([
        src_p[:E0].reshape(NS, G0 * K),
        jnp.pad(src_p[E0:].reshape(NS, G1 * K),
                ((0, 0), (0, (G0 - G1) * K))),
    ], axis=0)
    dst_t = jnp.concatenate([
        dst_p[:E0].reshape(NS, G0, K),
        jnp.pad(dst_p[E0:].reshape(NS, G1, K),
                ((0, 0), (0, G0 - G1), (0, 0)), constant_values=N),
    ], axis=0)
    offs = jnp.arange(CHUNKS, dtype=jnp.int32) * N
    src4 = src_t[:, None, :] + offs[None, :, None]
    dst_d = jnp.concatenate(
        [dst, jnp.full((E_PAD_D - E,), N, jnp.int32)]).reshape(NT, GD, K)

    ones128 = jnp.ones((K, 128), jnp.float32)
    zeros128 = jnp.zeros((ZERO_ROWS, 128), jnp.float32)

    deg_kernel, mp_kernel = _sc_kernels()
    deg_parts = deg_kernel(dst_d, ones128, zeros128)
    table = _tc1(x, h_cur, W, deg_parts)
    parts = mp_kernel(table.reshape(CHUNKS * N, 128), src4, dst_t, zeros128)
    h_next, c_next = _tc2(parts, table, deg_parts, c_cur, bias.reshape(2, 128))
    return (h_next, c_next)


# speedup vs baseline: 25.2040x; 1.0013x over previous
import functools

import jax
import jax.numpy as jnp
from jax import lax
from jax.experimental import pallas as pl
from jax.experimental.pallas import tpu as pltpu
from jax.experimental.pallas import tpu_sc as plsc

B = 2
N = 10000
E = 160000
D_IN = 128
D_H = 64
F = 4 * D_H

NC = 2
NS = 16
NT = NC * NS
K = 64
NBUF = 3
G0 = 123
G1 = 36
GD = 80
E0 = NS * G0 * K
E1 = NS * G1 * K
E_PAD = E0 + E1
E_PAD_D = NT * GD * K
ACC_ROWS = 10240
ZERO_ROWS = ACC_ROWS // NS
NBLK = 1000
CHUNKS = 4

@functools.lru_cache(maxsize=1)
def _sc_kernels():
    mesh = plsc.VectorSubcoreMesh(core_axis_name="c", subcore_axis_name="s",
                                  num_cores=NC, num_subcores=NS)

    @functools.partial(
        pl.kernel,
        out_type=jax.ShapeDtypeStruct((NC, ACC_ROWS, 128), jnp.float32),
        mesh=mesh,
        scratch_types=[
            pltpu.VMEM((GD, K), jnp.int32),
            pltpu.VMEM((K, 128), jnp.float32),
            pltpu.VMEM_SHARED((ACC_ROWS, 128), jnp.float32),
        ],
    )
    def deg_kernel(dst_hbm, ones_hbm, zeros_hbm, out_hbm, dst_idx, ones_v, acc):
        c = lax.axis_index("c")
        s = lax.axis_index("s")
        eb = c * NS + s
        pltpu.sync_copy(dst_hbm.at[eb], dst_idx)
        pltpu.sync_copy(ones_hbm, ones_v)
        pltpu.sync_copy(zeros_hbm, acc.at[pl.ds(s * ZERO_ROWS, ZERO_ROWS)])
        plsc.subcore_barrier()
        for g in range(GD):
            pltpu.sync_copy(ones_v, acc.at[dst_idx.at[g]], add=True)
        plsc.subcore_barrier()
        pltpu.sync_copy(acc.at[pl.ds(s * ZERO_ROWS, ZERO_ROWS)],
                        out_hbm.at[c, pl.ds(s * ZERO_ROWS, ZERO_ROWS)])

    @functools.partial(
        pl.kernel,
        out_type=jax.ShapeDtypeStruct((NC, CHUNKS, ACC_ROWS, 128), jnp.float32),
        mesh=mesh,
        scratch_types=[
            pltpu.VMEM((G0 * K,), jnp.int32),
            pltpu.VMEM((G0, K), jnp.int32),
        ] + [pltpu.VMEM((K, 128), jnp.float32) for _ in range(NBUF)]
          + [pltpu.VMEM_SHARED((ACC_ROWS, 128), jnp.float32)]
          + [pltpu.SemaphoreType.DMA for _ in range(NBUF)],
    )
    def mp_kernel(table_hbm, src_hbm, dst_hbm, zeros_hbm, out_hbm,
                  src_idx, dst_idx, *rest):
        bufs = rest[:NBUF]
        acc = rest[NBUF]
        sems = rest[NBUF + 1:]
        c = lax.axis_index("c")
        s = lax.axis_index("s")
        eb = c * NS + s
        ng = jnp.where(c == 0, G0, G1)
        pltpu.sync_copy(dst_hbm.at[eb], dst_idx)

        @pl.loop(0, CHUNKS)
        def _(ch):
            pltpu.sync_copy(zeros_hbm.at[pl.ds(0, K)], bufs[0])
            for j in range(ZERO_ROWS // K):
                pltpu.sync_copy(bufs[0],
                                acc.at[pl.ds(s * ZERO_ROWS + j * K, K)])
            pltpu.sync_copy(src_hbm.at[eb, ch], src_idx)
            plsc.subcore_barrier()

            for b in range(NBUF):
                pltpu.async_copy(table_hbm.at[src_idx.at[pl.ds(b * K, K)]],
                                 bufs[b], sems[b])

            @pl.loop(0, ng - NBUF, step=NBUF)
            def _(g):
                for b in range(NBUF):
                    pltpu.make_async_copy(table_hbm.at[src_idx.at[pl.ds(0, K)]],
                                          bufs[b], sems[b]).wait()
                    pltpu.sync_copy(bufs[b], acc.at[dst_idx.at[g + b]], add=True)
                    pltpu.async_copy(
                        table_hbm.at[src_idx.at[pl.ds((g + NBUF + b) * K, K)]],
                        bufs[b], sems[b])

            for b in range(NBUF):
                pltpu.make_async_copy(table_hbm.at[src_idx.at[pl.ds(0, K)]],
                                      bufs[b], sems[b]).wait()
                pltpu.sync_copy(bufs[b], acc.at[dst_idx.at[ng - NBUF + b]],
                                add=True)

            plsc.subcore_barrier()
            pltpu.sync_copy(acc.at[pl.ds(s * ZERO_ROWS, ZERO_ROWS)],
                            out_hbm.at[c, ch, pl.ds(s * ZERO_ROWS, ZERO_ROWS)])
            plsc.subcore_barrier()

    return deg_kernel, mp_kernel


def _tc1_body(x_ref, h_ref, w_ref, dp_ref, out_ref):
    xb = x_ref[0]
    hb = h_ref[0]
    xw = (jnp.dot(xb, w_ref[0:D_IN, :], preferred_element_type=jnp.float32)
          + jnp.dot(hb, w_ref[D_IN:, :], preferred_element_type=jnp.float32))
    deg = dp_ref[0, :, 0] + dp_ref[1, :, 0] + 1.0
    dinv = lax.rsqrt(deg)
    scaled = xw * dinv[:, None]
    out_ref[0] = scaled[:, 0:128]
    out_ref[1] = scaled[:, 128:256]


def _tc1(x, h_cur, w, deg_parts):
    return pl.pallas_call(
        _tc1_body,
        grid=(B, N // NBLK),
        in_specs=[
            pl.BlockSpec((1, NBLK, D_IN), lambda b, n: (b, n, 0)),
            pl.BlockSpec((1, NBLK, D_H), lambda b, n: (b, n, 0)),
            pl.BlockSpec((D_IN + D_H, F), lambda b, n: (0, 0)),
            pl.BlockSpec((NC, NBLK, 128), lambda b, n: (0, n, 0)),
        ],
        out_specs=pl.BlockSpec((2, NBLK, 128), lambda b, n: (b, n, 0)),
        out_shape=jax.ShapeDtypeStruct((CHUNKS, N, 128), jnp.float32),
    )(x, h_cur, w, deg_parts)


def _tc2_body(parts_ref, table_ref, dp_ref, c_ref, bias_ref, h_out, c_out):
    s0 = parts_ref[0, 0] + parts_ref[1, 0] + table_ref[0]
    s1 = parts_ref[0, 1] + parts_ref[1, 1] + table_ref[1]
    deg = dp_ref[0, :, 0] + dp_ref[1, :, 0] + 1.0
    dinv = lax.rsqrt(deg)
    a0 = s0 * dinv[:, None] + bias_ref[0:1, :]
    a1 = s1 * dinv[:, None] + bias_ref[1:2, :]
    gi = jax.nn.sigmoid(a0[:, 0:D_H])
    gf = jax.nn.sigmoid(a0[:, D_H:2 * D_H])
    go = jax.nn.sigmoid(a1[:, 0:D_H])
    gg = jnp.tanh(a1[:, D_H:2 * D_H])
    cn = gf * c_ref[0] + gi * gg
    c_out[0] = cn
    h_out[0] = go * jnp.tanh(cn)


def _tc2(parts, table, deg_parts, c_cur, bias2):
    return pl.pallas_call(
        _tc2_body,
        grid=(B, N // NBLK),
        in_specs=[
            pl.BlockSpec((NC, 2, NBLK, 128), lambda b, n: (0, b, n, 0)),
            pl.BlockSpec((2, NBLK, 128), lambda b, n: (b, n, 0)),
            pl.BlockSpec((NC, NBLK, 128), lambda b, n: (0, n, 0)),
            pl.BlockSpec((1, NBLK, D_H), lambda b, n: (b, n, 0)),
            pl.BlockSpec((2, 128), lambda b, n: (0, 0)),
        ],
        out_specs=[
            pl.BlockSpec((1, NBLK, D_H), lambda b, n: (b, n, 0)),
            pl.BlockSpec((1, NBLK, D_H), lambda b, n: (b, n, 0)),
        ],
        out_shape=[
            jax.ShapeDtypeStruct((B, N, D_H), jnp.float32),
            jax.ShapeDtypeStruct((B, N, D_H), jnp.float32),
        ],
    )(parts, table, deg_parts, c_cur, bias2)


def kernel(x, h_cur, c_cur, edge_index, W, bias):
    src = edge_index[0]
    dst = edge_index[1]
    npad = E_PAD - E
    src_p = jnp.concatenate([src, jnp.zeros((npad,), jnp.int32)])
    dst_p = jnp.concatenate([dst, jnp.full((npad,), N, jnp.int32)])
    src_t = jnp.concatenate([
        src_p[:E0].reshape(NS, G0 * K),
        jnp.pad(src_p[E0:].reshape(NS, G1 * K),
                ((0, 0), (0, (G0 - G1) * K))),
    ], axis=0)
    dst_t = jnp.concatenate([
        dst_p[:E0].reshape(NS, G0, K),
        jnp.pad(dst_p[E0:].reshape(NS, G1, K),
                ((0, 0), (0, G0 - G1), (0, 0)), constant_values=N),
    ], axis=0)
    offs = jnp.arange(CHUNKS, dtype=jnp.int32) * N
    src4 = src_t[:, None, :] + offs[None, :, None]
    dst_d = jnp.concatenate(
        [dst, jnp.full((E_PAD_D - E,), N, jnp.int32)]).reshape(NT, GD, K)

    ones128 = jnp.ones((K, 128), jnp.float32)
    zeros128 = jnp.zeros((ZERO_ROWS, 128), jnp.float32)

    deg_kernel, mp_kernel = _sc_kernels()
    deg_parts = deg_kernel(dst_d, ones128, zeros128)
    table = _tc1(x, h_cur, W, deg_parts)
    parts = mp_kernel(table.reshape(CHUNKS * N, 128), src4, dst_t, zeros128)
    h_next, c_next = _tc2(parts, table, deg_parts, c_cur, bias.reshape(2, 128))
    return (h_next, c_next)
